# baseline (speedup 1.0000x reference)
import jax
import jax.numpy as jnp
from jax.experimental import pallas as pl


def _add_kernel(x_ref, t_ref, o_ref):
    o_ref[...] = x_ref[...] + t_ref[...][None, :, None, :]


def kernel(x, table):
    B, T, N, C = x.shape
    TB = 8
    grid = (B, T // TB)
    return pl.pallas_call(
        _add_kernel,
        grid=grid,
        in_specs=[
            pl.BlockSpec((1, TB, N, C), lambda b, t: (b, t, 0, 0)),
            pl.BlockSpec((TB, C), lambda b, t: (t, 0)),
        ],
        out_specs=pl.BlockSpec((1, TB, N, C), lambda b, t: (b, t, 0, 0)),
        out_shape=jax.ShapeDtypeStruct(x.shape, x.dtype),
    )(x, table)
